# baseline (speedup 1.0000x reference)
import functools

import jax
import jax.numpy as jnp
from jax import lax
from jax.experimental import pallas as pl
from jax.experimental.pallas import tpu as pltpu
from jax.experimental.pallas import tpu_sc as plsc

N = 10000
E = 320000
IN_DIM = 128
NH = 64
D0 = 128
NC = 40
NCPAD = 64

NPAD = 10240
NB = 1024
NCORE = 2
NSUB = 16
NWORK = NCORE * NSUB
LANES = 16
EW = E // NWORK
NSLICE = NPAD // NSUB
EB = 8000
NEB = E // EB


def _mesh():
    return plsc.VectorSubcoreMesh(
        core_axis_name="c", subcore_axis_name="s",
        num_cores=NCORE, num_subcores=NSUB)


_SC_PARAMS = pltpu.CompilerParams(needs_layout_passes=False)


def _zero_f32(ref, n):
    def body(i, _):
        ref[pl.ds(i * LANES, LANES)] = jnp.zeros((LANES,), jnp.float32)
        return None
    lax.fori_loop(0, n // LANES, body, None)


ES = E // NSUB
EHALF = ES // NCORE


def _sc_b1_body(el_h, er_h, src_h, dst_h, alpha_h,
                srcb, dstb, elb, erb, pb, denl, tmp, acc, parts, denf):
    c = lax.axis_index("c")
    s = lax.axis_index("s")
    base = s * ES
    pltpu.sync_copy(src_h.at[pl.ds(base, ES)], srcb)
    pltpu.sync_copy(dst_h.at[pl.ds(base, ES)], dstb)
    pltpu.sync_copy(el_h, elb)
    pltpu.sync_copy(er_h, erb)
    _zero_f32(denl, NPAD)

    @plsc.parallel_loop(0, ES // LANES, unroll=4)
    def edge_chunk(i):
        i16 = pl.ds(i * LANES, LANES)
        s16 = srcb[i16]
        d16 = dstb[i16]
        a = plsc.load_gather(elb, [s16])
        b = plsc.load_gather(erb, [d16])
        e = a + b
        e = jnp.where(e >= 0.0, e, 0.2 * e)
        pv = jnp.exp(e)
        pb[i16] = pv
        plsc.addupdate_scatter(denl, [d16], pv)

    pltpu.sync_copy(denl, parts.at[s])
    plsc.subcore_barrier()

    _zero_f32(acc, NSLICE)

    def part_add(t, _):
        pltpu.sync_copy(parts.at[t, pl.ds(s * NSLICE, NSLICE)], tmp)

        def vec_add(k, _):
            k16 = pl.ds(k * LANES, LANES)
            acc[k16] = acc[k16] + tmp[k16]
            return None
        lax.fori_loop(0, NSLICE // LANES, vec_add, None)
        return None
    lax.fori_loop(0, NSUB, part_add, None)

    pltpu.sync_copy(acc, denf.at[pl.ds(s * NSLICE, NSLICE)])
    plsc.subcore_barrier()
    pltpu.sync_copy(denf, denl)

    hoff = c * EHALF

    @plsc.parallel_loop(0, EHALF // LANES, unroll=4)
    def alpha_chunk(i):
        i16 = pl.ds(hoff + i * LANES, LANES)
        d16 = dstb[i16]
        dg = plsc.load_gather(denl, [d16])
        pb[i16] = pb[i16] / dg

    pltpu.sync_copy(pb.at[pl.ds(hoff, EHALF)],
                    alpha_h.at[pl.ds(base + hoff, EHALF)])


@functools.cache
def _sc_b1():
    return pl.kernel(
        _sc_b1_body,
        out_type=jax.ShapeDtypeStruct((E,), jnp.float32),
        mesh=_mesh(),
        scratch_types=[
            pltpu.VMEM((ES,), jnp.int32),
            pltpu.VMEM((ES,), jnp.int32),
            pltpu.VMEM((NPAD,), jnp.float32),
            pltpu.VMEM((NPAD,), jnp.float32),
            pltpu.VMEM((ES,), jnp.float32),
            pltpu.VMEM((NPAD,), jnp.float32),
            pltpu.VMEM((NSLICE,), jnp.float32),
            pltpu.VMEM((NSLICE,), jnp.float32),
            pltpu.VMEM_SHARED((NSUB, NPAD), jnp.float32),
            pltpu.VMEM_SHARED((NPAD,), jnp.float32),
        ],
        compiler_params=_SC_PARAMS,
    )


def _sc_c_body(r_rows, src_h, dst_h, alpha_h, feat_h, out_h,
               sems, srcb, dstb, abuf, *rowbufs):
    frows = rowbufs[:r_rows]
    orows = rowbufs[r_rows:]
    c = lax.axis_index("c")
    s = lax.axis_index("s")
    w = c * NSUB + s
    r0 = w * r_rows

    def start(b, q):
        eb = b * EB
        pltpu.async_copy(src_h.at[pl.ds(eb, EB)], srcb[q], sems.at[q, 0])
        pltpu.async_copy(dst_h.at[pl.ds(eb, EB)], dstb[q], sems.at[q, 1])
        pltpu.async_copy(alpha_h.at[pl.ds(eb, EB)], abuf[q], sems.at[q, 2])

    def wait(b, q):
        eb = b * EB
        pltpu.make_async_copy(src_h.at[pl.ds(eb, EB)], srcb[q],
                              sems.at[q, 0]).wait()
        pltpu.make_async_copy(dst_h.at[pl.ds(eb, EB)], dstb[q],
                              sems.at[q, 1]).wait()
        pltpu.make_async_copy(alpha_h.at[pl.ds(eb, EB)], abuf[q],
                              sems.at[q, 2]).wait()

    def compute(q):
        @plsc.parallel_loop(0, EB // LANES, unroll=10)
        def chunk(k):
            i16 = pl.ds(k * LANES, LANES)
            s16 = srcb[q][i16]
            d16 = dstb[q][i16]
            al = abuf[q][i16]
            for j in range(r_rows):
                v = plsc.load_gather(frows[j], [s16])
                plsc.addupdate_scatter(orows[j], [d16], v * al)

    for j in range(r_rows):
        pltpu.sync_copy(feat_h.at[r0 + j], frows[j])
        _zero_f32(orows[j], NPAD)

    start(0, 0)
    start(1, 1)

    def edge_superblock(i, _):
        b0 = i * 2
        wait(b0, 0)
        compute(0)

        @pl.when(b0 + 2 < NEB)
        def _():
            start(b0 + 2, 0)
        wait(b0 + 1, 1)
        compute(1)

        @pl.when(b0 + 3 < NEB)
        def _():
            start(b0 + 3, 1)
        return None
    lax.fori_loop(0, NEB // 2, edge_superblock, None)

    for j in range(r_rows):
        pltpu.sync_copy(orows[j], out_h.at[r0 + j])


@functools.cache
def _sc_c(dpad):
    r_rows = dpad // NWORK
    return pl.kernel(
        functools.partial(_sc_c_body, r_rows),
        out_type=jax.ShapeDtypeStruct((dpad, NPAD), jnp.float32),
        mesh=_mesh(),
        scratch_types=[
            pltpu.SemaphoreType.DMA((2, 3)),
            [pltpu.VMEM((EB,), jnp.int32) for _ in range(2)],
            [pltpu.VMEM((EB,), jnp.int32) for _ in range(2)],
            [pltpu.VMEM((EB,), jnp.float32) for _ in range(2)],
        ] + [pltpu.VMEM((NPAD,), jnp.float32) for _ in range(2 * r_rows)],
        compiler_params=_SC_PARAMS,
    )


_HIGH = jax.lax.Precision.HIGHEST


def _dotT(a, b):
    return jnp.dot(a, b, precision=_HIGH, preferred_element_type=jnp.float32)


def _elrer(feat, alc, arc):
    el = jnp.sum(feat * alc, axis=0, keepdims=True)
    er = jnp.sum(feat * arc, axis=0, keepdims=True)
    return el, er


def _tc_pre_body(xt, z1, wxt, bx1, w0t, al0, ar0, h_o, f_o, el_o, er_o):
    x = xt[:, :]
    h0 = jnp.maximum(_dotT(wxt[:, :], x) + bx1[:, :], 0.0)
    zz = jnp.broadcast_to(z1[:, :], (NH, x.shape[1]))
    hcat = jnp.concatenate([h0, zz], axis=0)
    nrm = jnp.sqrt(jnp.sum(hcat * hcat, axis=0, keepdims=True)) + 1e-6
    h = hcat / nrm
    feat = _dotT(w0t[:, :], h)
    el, er = _elrer(feat, al0[:, :], ar0[:, :])
    h_o[:, :] = h
    f_o[:, :] = feat
    el_o[:, :] = el
    er_o[:, :] = er


def _elu(x):
    return jnp.where(x > 0.0, x, jnp.exp(x) - 1.0)


def _tc_mid1_body(o0t, w1t, al1, ar1, h_o, f_o, el_o, er_o):
    h1 = _elu(o0t[:, :])
    feat = _dotT(w1t[:, :], h1)
    el, er = _elrer(feat, al1[:, :], ar1[:, :])
    h_o[:, :] = h1
    f_o[:, :] = feat
    el_o[:, :] = el
    er_o[:, :] = er


def _tc_mid2_body(o1t, h1t, w2t, al2, ar2, h_o, f_o, el_o, er_o):
    h2 = _elu(o1t[:, :] + h1t[:, :])
    feat = _dotT(w2t[:, :], h2)
    el, er = _elrer(feat, al2[:, :], ar2[:, :])
    h_o[:, :] = h2
    f_o[:, :] = feat
    el_o[:, :] = el
    er_o[:, :] = er


def _tc_fin_body(o2t, h2t, rw2t, lg_o):
    lg_o[:, :] = o2t[:, :] + _dotT(rw2t[:, :], h2t[:, :])


def _col_spec(d):
    return pl.BlockSpec((d, NB), lambda j: (0, j))


def _full_spec(shape):
    return pl.BlockSpec(shape, lambda j: tuple(0 for _ in shape))


@functools.cache
def _tc_pre():
    return pl.pallas_call(
        _tc_pre_body,
        grid=(NPAD // NB,),
        in_specs=[
            _col_spec(IN_DIM),
            _full_spec((NH, 1)), _full_spec((NH, IN_DIM)), _full_spec((NH, 1)),
            _full_spec((D0, D0)), _full_spec((D0, 1)), _full_spec((D0, 1)),
        ],
        out_specs=[_col_spec(D0), _col_spec(D0), _col_spec(1), _col_spec(1)],
        out_shape=[
            jax.ShapeDtypeStruct((D0, NPAD), jnp.float32),
            jax.ShapeDtypeStruct((D0, NPAD), jnp.float32),
            jax.ShapeDtypeStruct((1, NPAD), jnp.float32),
            jax.ShapeDtypeStruct((1, NPAD), jnp.float32),
        ],
    )


@functools.cache
def _tc_mid1():
    return pl.pallas_call(
        _tc_mid1_body,
        grid=(NPAD // NB,),
        in_specs=[
            _col_spec(D0),
            _full_spec((D0, D0)), _full_spec((D0, 1)), _full_spec((D0, 1)),
        ],
        out_specs=[_col_spec(D0), _col_spec(D0), _col_spec(1), _col_spec(1)],
        out_shape=[
            jax.ShapeDtypeStruct((D0, NPAD), jnp.float32),
            jax.ShapeDtypeStruct((D0, NPAD), jnp.float32),
            jax.ShapeDtypeStruct((1, NPAD), jnp.float32),
            jax.ShapeDtypeStruct((1, NPAD), jnp.float32),
        ],
    )


@functools.cache
def _tc_mid2():
    return pl.pallas_call(
        _tc_mid2_body,
        grid=(NPAD // NB,),
        in_specs=[
            _col_spec(D0), _col_spec(D0),
            _full_spec((NCPAD, D0)), _full_spec((NCPAD, 1)), _full_spec((NCPAD, 1)),
        ],
        out_specs=[_col_spec(D0), _col_spec(NCPAD), _col_spec(1), _col_spec(1)],
        out_shape=[
            jax.ShapeDtypeStruct((D0, NPAD), jnp.float32),
            jax.ShapeDtypeStruct((NCPAD, NPAD), jnp.float32),
            jax.ShapeDtypeStruct((1, NPAD), jnp.float32),
            jax.ShapeDtypeStruct((1, NPAD), jnp.float32),
        ],
    )


@functools.cache
def _tc_fin():
    return pl.pallas_call(
        _tc_fin_body,
        grid=(NPAD // NB,),
        in_specs=[
            _col_spec(NCPAD), _col_spec(D0), _full_spec((NCPAD, D0)),
        ],
        out_specs=_col_spec(NCPAD),
        out_shape=jax.ShapeDtypeStruct((NCPAD, NPAD), jnp.float32),
    )


def kernel(inputs, z, edge_index, Wx, bx, W0, al0, ar0, W1, al1, ar1,
           W2, al2, ar2, resW2):
    src = edge_index[0]
    dst = edge_index[1]

    xt = jnp.pad(inputs, ((0, NPAD - N), (0, 0))).T
    z1 = z[:, None]
    bx1 = bx[:, None]
    wxt = Wx.T
    w0t = W0.T
    w1t = W1.T
    w2t = jnp.pad(W2, ((0, 0), (0, NCPAD - NC))).T
    rw2t = jnp.pad(resW2, ((0, 0), (0, NCPAD - NC))).T
    al0c = al0.reshape(D0, 1)
    ar0c = ar0.reshape(D0, 1)
    al1c = al1.reshape(D0, 1)
    ar1c = ar1.reshape(D0, 1)
    al2c = jnp.pad(al2.reshape(NC, 1), ((0, NCPAD - NC), (0, 0)))
    ar2c = jnp.pad(ar2.reshape(NC, 1), ((0, NCPAD - NC), (0, 0)))

    ht, f0, el0, er0 = _tc_pre()(xt, z1, wxt, bx1, w0t, al0c, ar0c)
    a0 = _sc_b1()(el0.reshape(NPAD), er0.reshape(NPAD), src, dst)
    o0 = _sc_c(D0)(src, dst, a0, f0)

    h1, f1, el1, er1 = _tc_mid1()(o0, w1t, al1c, ar1c)
    a1 = _sc_b1()(el1.reshape(NPAD), er1.reshape(NPAD), src, dst)
    o1 = _sc_c(D0)(src, dst, a1, f1)

    h2, f2, el2, er2 = _tc_mid2()(o1, h1, w2t, al2c, ar2c)
    a2 = _sc_b1()(el2.reshape(NPAD), er2.reshape(NPAD), src, dst)
    o2 = _sc_c(NCPAD)(src, dst, a2, f2)

    lgt = _tc_fin()(o2, h2, rw2t)
    return lgt[:NC, :N].T


# speedup vs baseline: 1.1141x; 1.1141x over previous
import functools

import jax
import jax.numpy as jnp
from jax import lax
from jax.experimental import pallas as pl
from jax.experimental.pallas import tpu as pltpu
from jax.experimental.pallas import tpu_sc as plsc

N = 10000
E = 320000
IN_DIM = 128
NH = 64
D0 = 128
NC = 40
NCPAD = 64

NPAD = 10240
NB = 1024
NCORE = 2
NSUB = 16
NWORK = NCORE * NSUB
LANES = 16
EW = E // NWORK
NSLICE = NPAD // NSUB
EB = 8000
NEB = E // EB


def _mesh():
    return plsc.VectorSubcoreMesh(
        core_axis_name="c", subcore_axis_name="s",
        num_cores=NCORE, num_subcores=NSUB)


_SC_PARAMS = pltpu.CompilerParams(needs_layout_passes=False)


def _zero_f32(ref, n):
    def body(i, _):
        ref[pl.ds(i * LANES, LANES)] = jnp.zeros((LANES,), jnp.float32)
        return None
    lax.fori_loop(0, n // LANES, body, None)


ES = E // NSUB
EHALF = ES // NCORE


def _sc_b1_body(el_h, er_h, src_h, dst_h, alpha_h,
                srcb, dstb, elb, erb, pb, denl, tmp, acc, parts, denf):
    c = lax.axis_index("c")
    s = lax.axis_index("s")
    base = s * ES
    pltpu.sync_copy(src_h.at[pl.ds(base, ES)], srcb)
    pltpu.sync_copy(dst_h.at[pl.ds(base, ES)], dstb)
    pltpu.sync_copy(el_h, elb)
    pltpu.sync_copy(er_h, erb)
    _zero_f32(denl, NPAD)

    @plsc.parallel_loop(0, ES // LANES, unroll=4)
    def edge_chunk(i):
        i16 = pl.ds(i * LANES, LANES)
        s16 = srcb[i16]
        d16 = dstb[i16]
        a = plsc.load_gather(elb, [s16])
        b = plsc.load_gather(erb, [d16])
        e = a + b
        e = jnp.where(e >= 0.0, e, 0.2 * e)
        pv = jnp.exp(e)
        pb[i16] = pv
        plsc.addupdate_scatter(denl, [d16], pv)

    pltpu.sync_copy(denl, parts.at[s])
    plsc.subcore_barrier()

    _zero_f32(acc, NSLICE)

    def part_add(t, _):
        pltpu.sync_copy(parts.at[t, pl.ds(s * NSLICE, NSLICE)], tmp)

        def vec_add(k, _):
            k16 = pl.ds(k * LANES, LANES)
            acc[k16] = acc[k16] + tmp[k16]
            return None
        lax.fori_loop(0, NSLICE // LANES, vec_add, None)
        return None
    lax.fori_loop(0, NSUB, part_add, None)

    pltpu.sync_copy(acc, denf.at[pl.ds(s * NSLICE, NSLICE)])
    plsc.subcore_barrier()
    pltpu.sync_copy(denf, denl)

    hoff = c * EHALF

    @plsc.parallel_loop(0, EHALF // LANES, unroll=4)
    def alpha_chunk(i):
        i16 = pl.ds(hoff + i * LANES, LANES)
        d16 = dstb[i16]
        dg = plsc.load_gather(denl, [d16])
        pb[i16] = pb[i16] / dg

    pltpu.sync_copy(pb.at[pl.ds(hoff, EHALF)],
                    alpha_h.at[pl.ds(base + hoff, EHALF)])


@functools.cache
def _sc_b1():
    return pl.kernel(
        _sc_b1_body,
        out_type=jax.ShapeDtypeStruct((E,), jnp.float32),
        mesh=_mesh(),
        scratch_types=[
            pltpu.VMEM((ES,), jnp.int32),
            pltpu.VMEM((ES,), jnp.int32),
            pltpu.VMEM((NPAD,), jnp.float32),
            pltpu.VMEM((NPAD,), jnp.float32),
            pltpu.VMEM((ES,), jnp.float32),
            pltpu.VMEM((NPAD,), jnp.float32),
            pltpu.VMEM((NSLICE,), jnp.float32),
            pltpu.VMEM((NSLICE,), jnp.float32),
            pltpu.VMEM_SHARED((NSUB, NPAD), jnp.float32),
            pltpu.VMEM_SHARED((NPAD,), jnp.float32),
        ],
        compiler_params=_SC_PARAMS,
    )


def _sc_c_body(r_rows, dpad, src_h, dst_h, alpha_h, feat_h, out_h,
               sems, srcb, dstb, abuf, *rowbufs):
    frows = rowbufs[:r_rows]
    orows = rowbufs[r_rows:]
    c = lax.axis_index("c")
    s = lax.axis_index("s")
    w = c * NSUB + s
    r0 = w * r_rows

    def start(b, q):
        eb = b * EB
        pltpu.async_copy(src_h.at[pl.ds(eb, EB)], srcb[q], sems.at[q, 0])
        pltpu.async_copy(dst_h.at[pl.ds(eb, EB)], dstb[q], sems.at[q, 1])
        pltpu.async_copy(alpha_h.at[pl.ds(eb, EB)], abuf[q], sems.at[q, 2])

    def wait(b, q):
        eb = b * EB
        pltpu.make_async_copy(src_h.at[pl.ds(eb, EB)], srcb[q],
                              sems.at[q, 0]).wait()
        pltpu.make_async_copy(dst_h.at[pl.ds(eb, EB)], dstb[q],
                              sems.at[q, 1]).wait()
        pltpu.make_async_copy(alpha_h.at[pl.ds(eb, EB)], abuf[q],
                              sems.at[q, 2]).wait()

    def compute(q):
        @plsc.parallel_loop(0, EB // LANES, unroll=4)
        def chunk(k):
            i16 = pl.ds(k * LANES, LANES)
            s16 = srcb[q][i16]
            d16 = dstb[q][i16]
            al = abuf[q][i16]
            for j in range(r_rows):
                v = plsc.load_gather(frows[j], [s16])
                lo = plsc.bitcast(v << 16, jnp.float32)
                hi = plsc.bitcast(v & jnp.int32(-65536), jnp.float32)
                plsc.addupdate_scatter(orows[2 * j], [d16], lo * al)
                plsc.addupdate_scatter(orows[2 * j + 1], [d16], hi * al)

    for j in range(r_rows):
        pltpu.sync_copy(feat_h.at[r0 + j], frows[j])
        _zero_f32(orows[2 * j], NPAD)
        _zero_f32(orows[2 * j + 1], NPAD)

    start(0, 0)
    start(1, 1)

    def edge_superblock(i, _):
        b0 = i * 2
        wait(b0, 0)
        compute(0)

        @pl.when(b0 + 2 < NEB)
        def _():
            start(b0 + 2, 0)
        wait(b0 + 1, 1)
        compute(1)

        @pl.when(b0 + 3 < NEB)
        def _():
            start(b0 + 3, 1)
        return None
    lax.fori_loop(0, NEB // 2, edge_superblock, None)

    for j in range(r_rows):
        pltpu.sync_copy(orows[2 * j], out_h.at[r0 + j])
        pltpu.sync_copy(orows[2 * j + 1], out_h.at[r0 + j + dpad // 2])


@functools.cache
def _sc_c(dpad):
    r_rows = dpad // 2 // NWORK
    return pl.kernel(
        functools.partial(_sc_c_body, r_rows, dpad),
        out_type=jax.ShapeDtypeStruct((dpad, NPAD), jnp.float32),
        mesh=_mesh(),
        scratch_types=[
            pltpu.SemaphoreType.DMA((2, 3)),
            [pltpu.VMEM((EB,), jnp.int32) for _ in range(2)],
            [pltpu.VMEM((EB,), jnp.int32) for _ in range(2)],
            [pltpu.VMEM((EB,), jnp.float32) for _ in range(2)],
        ] + [pltpu.VMEM((NPAD,), jnp.int32) for _ in range(r_rows)]
          + [pltpu.VMEM((NPAD,), jnp.float32) for _ in range(2 * r_rows)],
        compiler_params=_SC_PARAMS,
    )


_HIGH = jax.lax.Precision.HIGHEST


def _dotT(a, b):
    return jnp.dot(a, b, precision=_HIGH, preferred_element_type=jnp.float32)


def _elrer(feat, alc, arc):
    el = jnp.sum(feat * alc, axis=0, keepdims=True)
    er = jnp.sum(feat * arc, axis=0, keepdims=True)
    return el, er


def _pack_rows(feat):
    half = feat.shape[0] // 2
    tu = jax.lax.bitcast_convert_type(
        feat[:half].astype(jnp.bfloat16), jnp.uint16).astype(jnp.uint32)
    bu = jax.lax.bitcast_convert_type(
        feat[half:].astype(jnp.bfloat16), jnp.uint16).astype(jnp.uint32)
    return jax.lax.bitcast_convert_type(tu | (bu << 16), jnp.int32)


def _tc_pre_body(xt, z1, wxt, bx1, w0t, al0, ar0, h_o, f_o, el_o, er_o):
    x = xt[:, :]
    h0 = jnp.maximum(_dotT(wxt[:, :], x) + bx1[:, :], 0.0)
    zz = jnp.broadcast_to(z1[:, :], (NH, x.shape[1]))
    hcat = jnp.concatenate([h0, zz], axis=0)
    nrm = jnp.sqrt(jnp.sum(hcat * hcat, axis=0, keepdims=True)) + 1e-6
    h = hcat / nrm
    feat = _dotT(w0t[:, :], h)
    el, er = _elrer(feat, al0[:, :], ar0[:, :])
    h_o[:, :] = h
    f_o[:, :] = _pack_rows(feat)
    el_o[:, :] = el
    er_o[:, :] = er


def _elu(x):
    return jnp.where(x > 0.0, x, jnp.exp(x) - 1.0)


def _tc_mid1_body(o0t, w1t, al1, ar1, h_o, f_o, el_o, er_o):
    h1 = _elu(o0t[:, :])
    feat = _dotT(w1t[:, :], h1)
    el, er = _elrer(feat, al1[:, :], ar1[:, :])
    h_o[:, :] = h1
    f_o[:, :] = _pack_rows(feat)
    el_o[:, :] = el
    er_o[:, :] = er


def _tc_mid2_body(o1t, h1t, w2t, al2, ar2, h_o, f_o, el_o, er_o):
    h2 = _elu(o1t[:, :] + h1t[:, :])
    feat = _dotT(w2t[:, :], h2)
    el, er = _elrer(feat, al2[:, :], ar2[:, :])
    h_o[:, :] = h2
    f_o[:, :] = _pack_rows(feat)
    el_o[:, :] = el
    er_o[:, :] = er


def _tc_fin_body(o2t, h2t, rw2t, lg_o):
    lg_o[:, :] = o2t[:, :] + _dotT(rw2t[:, :], h2t[:, :])


def _col_spec(d):
    return pl.BlockSpec((d, NB), lambda j: (0, j))


def _full_spec(shape):
    return pl.BlockSpec(shape, lambda j: tuple(0 for _ in shape))


@functools.cache
def _tc_pre():
    return pl.pallas_call(
        _tc_pre_body,
        grid=(NPAD // NB,),
        in_specs=[
            _col_spec(IN_DIM),
            _full_spec((NH, 1)), _full_spec((NH, IN_DIM)), _full_spec((NH, 1)),
            _full_spec((D0, D0)), _full_spec((D0, 1)), _full_spec((D0, 1)),
        ],
        out_specs=[_col_spec(D0), _col_spec(D0 // 2), _col_spec(1), _col_spec(1)],
        out_shape=[
            jax.ShapeDtypeStruct((D0, NPAD), jnp.float32),
            jax.ShapeDtypeStruct((D0 // 2, NPAD), jnp.int32),
            jax.ShapeDtypeStruct((1, NPAD), jnp.float32),
            jax.ShapeDtypeStruct((1, NPAD), jnp.float32),
        ],
    )


@functools.cache
def _tc_mid1():
    return pl.pallas_call(
        _tc_mid1_body,
        grid=(NPAD // NB,),
        in_specs=[
            _col_spec(D0),
            _full_spec((D0, D0)), _full_spec((D0, 1)), _full_spec((D0, 1)),
        ],
        out_specs=[_col_spec(D0), _col_spec(D0 // 2), _col_spec(1), _col_spec(1)],
        out_shape=[
            jax.ShapeDtypeStruct((D0, NPAD), jnp.float32),
            jax.ShapeDtypeStruct((D0 // 2, NPAD), jnp.int32),
            jax.ShapeDtypeStruct((1, NPAD), jnp.float32),
            jax.ShapeDtypeStruct((1, NPAD), jnp.float32),
        ],
    )


@functools.cache
def _tc_mid2():
    return pl.pallas_call(
        _tc_mid2_body,
        grid=(NPAD // NB,),
        in_specs=[
            _col_spec(D0), _col_spec(D0),
            _full_spec((NCPAD, D0)), _full_spec((NCPAD, 1)), _full_spec((NCPAD, 1)),
        ],
        out_specs=[_col_spec(D0), _col_spec(NCPAD // 2), _col_spec(1), _col_spec(1)],
        out_shape=[
            jax.ShapeDtypeStruct((D0, NPAD), jnp.float32),
            jax.ShapeDtypeStruct((NCPAD // 2, NPAD), jnp.int32),
            jax.ShapeDtypeStruct((1, NPAD), jnp.float32),
            jax.ShapeDtypeStruct((1, NPAD), jnp.float32),
        ],
    )


@functools.cache
def _tc_fin():
    return pl.pallas_call(
        _tc_fin_body,
        grid=(NPAD // NB,),
        in_specs=[
            _col_spec(NCPAD), _col_spec(D0), _full_spec((NCPAD, D0)),
        ],
        out_specs=_col_spec(NCPAD),
        out_shape=jax.ShapeDtypeStruct((NCPAD, NPAD), jnp.float32),
    )


def kernel(inputs, z, edge_index, Wx, bx, W0, al0, ar0, W1, al1, ar1,
           W2, al2, ar2, resW2):
    src = edge_index[0]
    dst = edge_index[1]

    xt = jnp.pad(inputs, ((0, NPAD - N), (0, 0))).T
    z1 = z[:, None]
    bx1 = bx[:, None]
    wxt = Wx.T
    w0t = W0.T
    w1t = W1.T
    w2t = jnp.pad(W2, ((0, 0), (0, NCPAD - NC))).T
    rw2t = jnp.pad(resW2, ((0, 0), (0, NCPAD - NC))).T
    al0c = al0.reshape(D0, 1)
    ar0c = ar0.reshape(D0, 1)
    al1c = al1.reshape(D0, 1)
    ar1c = ar1.reshape(D0, 1)
    al2c = jnp.pad(al2.reshape(NC, 1), ((0, NCPAD - NC), (0, 0)))
    ar2c = jnp.pad(ar2.reshape(NC, 1), ((0, NCPAD - NC), (0, 0)))

    ht, f0, el0, er0 = _tc_pre()(xt, z1, wxt, bx1, w0t, al0c, ar0c)
    a0 = _sc_b1()(el0.reshape(NPAD), er0.reshape(NPAD), src, dst)
    o0 = _sc_c(D0)(src, dst, a0, f0)

    h1, f1, el1, er1 = _tc_mid1()(o0, w1t, al1c, ar1c)
    a1 = _sc_b1()(el1.reshape(NPAD), er1.reshape(NPAD), src, dst)
    o1 = _sc_c(D0)(src, dst, a1, f1)

    h2, f2, el2, er2 = _tc_mid2()(o1, h1, w2t, al2c, ar2c)
    a2 = _sc_b1()(el2.reshape(NPAD), er2.reshape(NPAD), src, dst)
    o2 = _sc_c(NCPAD)(src, dst, a2, f2)

    lgt = _tc_fin()(o2, h2, rw2t)
    return lgt[:NC, :N].T


# speedup vs baseline: 1.1333x; 1.0172x over previous
import functools

import jax
import jax.numpy as jnp
from jax import lax
from jax.experimental import pallas as pl
from jax.experimental.pallas import tpu as pltpu
from jax.experimental.pallas import tpu_sc as plsc

N = 10000
E = 320000
IN_DIM = 128
NH = 64
D0 = 128
NC = 40
NCPAD = 64

NPAD = 10240
NB = 1024
NCORE = 2
NSUB = 16
NWORK = NCORE * NSUB
LANES = 16
EW = E // NWORK
NSLICE = NPAD // NSUB
EB = 8000
NEB = E // EB


def _mesh():
    return plsc.VectorSubcoreMesh(
        core_axis_name="c", subcore_axis_name="s",
        num_cores=NCORE, num_subcores=NSUB)


_SC_PARAMS = pltpu.CompilerParams(needs_layout_passes=False)


def _zero_f32(ref, n):
    def body(i, _):
        ref[pl.ds(i * LANES, LANES)] = jnp.zeros((LANES,), jnp.float32)
        return None
    lax.fori_loop(0, n // LANES, body, None)


ES = E // NSUB


def _sc_g_body(r_rows, dpad, el_h, er_h, src_h, dst_h, feat_h, out_h, aux_h,
               parts_h, sems, srcb, dstb, pb, elb, erb, denl, redb, acc, *rest):
    frows = rest[:r_rows]
    denf = rest[r_rows]
    c = lax.axis_index("c")
    s = lax.axis_index("s")
    r0 = (c * NSUB + s) * r_rows

    for j in range(r_rows):
        pltpu.async_copy(feat_h.at[r0 + j], frows[j], sems.at[2, 0])

    base = s * ES
    pltpu.sync_copy(src_h.at[pl.ds(base, ES)], srcb)
    pltpu.sync_copy(dst_h.at[pl.ds(base, ES)], dstb)
    pltpu.sync_copy(el_h, elb)
    pltpu.sync_copy(er_h, erb)
    _zero_f32(denl, NPAD)

    @plsc.parallel_loop(0, ES // LANES, unroll=4)
    def edge_chunk(i):
        i16 = pl.ds(i * LANES, LANES)
        s16 = srcb[i16]
        d16 = dstb[i16]
        a = plsc.load_gather(elb, [s16])
        b = plsc.load_gather(erb, [d16])
        e = a + b
        e = jnp.where(e >= 0.0, e, 0.2 * e)
        pv = jnp.exp(e)
        pb[i16] = pv
        plsc.addupdate_scatter(denl, [d16], pv)

    pltpu.sync_copy(denl, parts_h.at[pl.ds((c * NSUB + s) * NPAD, NPAD)])
    plsc.subcore_barrier()

    for t in range(NSUB):
        pltpu.async_copy(
            parts_h.at[pl.ds((c * NSUB + t) * NPAD + s * NSLICE, NSLICE)],
            redb.at[pl.ds(t * NSLICE, NSLICE)], sems.at[3, 0])
    for t in range(NSUB):
        pltpu.make_async_copy(
            parts_h.at[pl.ds((c * NSUB + t) * NPAD + s * NSLICE, NSLICE)],
            redb.at[pl.ds(t * NSLICE, NSLICE)], sems.at[3, 0]).wait()
    _zero_f32(acc, NSLICE)

    def part_add(t, _):
        def vec_add(k, _):
            k16 = pl.ds(k * LANES, LANES)
            acc[k16] = acc[k16] + redb[pl.ds(t * NSLICE + k * LANES, LANES)]
            return None
        lax.fori_loop(0, NSLICE // LANES, vec_add, None)
        return None
    lax.fori_loop(0, NSUB, part_add, None)

    pltpu.sync_copy(acc, denf.at[pl.ds(s * NSLICE, NSLICE)])
    plsc.subcore_barrier()
    pltpu.sync_copy(denf, denl)

    @plsc.parallel_loop(0, ES // LANES, unroll=4)
    def alpha_chunk(i):
        i16 = pl.ds(i * LANES, LANES)
        d16 = dstb[i16]
        dg = plsc.load_gather(denl, [d16])
        pb[i16] = pb[i16] / dg

    pltpu.sync_copy(pb, aux_h.at[pl.ds(c * E + base, ES)])

    for j in range(r_rows):
        pltpu.make_async_copy(feat_h.at[r0 + j], frows[j],
                              sems.at[2, 0]).wait()
    orows = [elb, erb, denl, redb][:2 * r_rows]
    for o in orows:
        _zero_f32(o, NPAD)
    plsc.subcore_barrier()

    def start(b, q):
        eb = b * EB
        qb = q * EB
        pltpu.async_copy(src_h.at[pl.ds(eb, EB)],
                         srcb.at[pl.ds(qb, EB)], sems.at[q, 0])
        pltpu.async_copy(dst_h.at[pl.ds(eb, EB)],
                         dstb.at[pl.ds(qb, EB)], sems.at[q, 1])
        pltpu.async_copy(aux_h.at[pl.ds(c * E + eb, EB)],
                         pb.at[pl.ds(qb, EB)], sems.at[q, 2])

    def wait(b, q):
        eb = b * EB
        qb = q * EB
        pltpu.make_async_copy(src_h.at[pl.ds(eb, EB)],
                              srcb.at[pl.ds(qb, EB)], sems.at[q, 0]).wait()
        pltpu.make_async_copy(dst_h.at[pl.ds(eb, EB)],
                              dstb.at[pl.ds(qb, EB)], sems.at[q, 1]).wait()
        pltpu.make_async_copy(aux_h.at[pl.ds(c * E + eb, EB)],
                              pb.at[pl.ds(qb, EB)], sems.at[q, 2]).wait()

    def compute(q):
        qb = q * EB

        @plsc.parallel_loop(0, EB // LANES, unroll=4)
        def chunk(k):
            i16 = pl.ds(qb + k * LANES, LANES)
            s16 = srcb[i16]
            d16 = dstb[i16]
            al = pb[i16]
            for j in range(r_rows):
                v = plsc.load_gather(frows[j], [s16])
                lo = plsc.bitcast(v << 16, jnp.float32)
                hi = plsc.bitcast(v & jnp.int32(-65536), jnp.float32)
                plsc.addupdate_scatter(orows[2 * j], [d16], lo * al)
                plsc.addupdate_scatter(orows[2 * j + 1], [d16], hi * al)

    start(0, 0)
    start(1, 1)

    def edge_superblock(i, _):
        b0 = i * 2
        wait(b0, 0)
        compute(0)

        @pl.when(b0 + 2 < NEB)
        def _():
            start(b0 + 2, 0)
        wait(b0 + 1, 1)
        compute(1)

        @pl.when(b0 + 3 < NEB)
        def _():
            start(b0 + 3, 1)
        return None
    lax.fori_loop(0, NEB // 2, edge_superblock, None)

    for j in range(r_rows):
        pltpu.sync_copy(orows[2 * j], out_h.at[r0 + j])
        pltpu.sync_copy(orows[2 * j + 1], out_h.at[r0 + j + dpad // 2])


@functools.cache
def _sc_g(dpad):
    r_rows = dpad // 2 // NWORK
    return pl.kernel(
        functools.partial(_sc_g_body, r_rows, dpad),
        out_type=[
            jax.ShapeDtypeStruct((dpad, NPAD), jnp.float32),
            jax.ShapeDtypeStruct((NCORE * E,), jnp.float32),
            jax.ShapeDtypeStruct((NCORE * NSUB * NPAD,), jnp.float32),
        ],
        mesh=_mesh(),
        scratch_types=[
            pltpu.SemaphoreType.DMA((4, 3)),
            pltpu.VMEM((ES,), jnp.int32),
            pltpu.VMEM((ES,), jnp.int32),
            pltpu.VMEM((ES,), jnp.float32),
            pltpu.VMEM((NPAD,), jnp.float32),
            pltpu.VMEM((NPAD,), jnp.float32),
            pltpu.VMEM((NPAD,), jnp.float32),
            pltpu.VMEM((NPAD,), jnp.float32),
            pltpu.VMEM((NSLICE,), jnp.float32),
        ] + [pltpu.VMEM((NPAD,), jnp.int32) for _ in range(r_rows)]
          + [
            pltpu.VMEM_SHARED((NPAD,), jnp.float32),
        ],
        compiler_params=_SC_PARAMS,
    )


_HIGH = jax.lax.Precision.HIGHEST


def _dotT(a, b):
    return jnp.dot(a, b, precision=_HIGH, preferred_element_type=jnp.float32)


def _elrer(feat, alc, arc):
    el = jnp.sum(feat * alc, axis=0, keepdims=True)
    er = jnp.sum(feat * arc, axis=0, keepdims=True)
    return el, er


def _pack_rows(feat):
    half = feat.shape[0] // 2
    tu = jax.lax.bitcast_convert_type(
        feat[:half].astype(jnp.bfloat16), jnp.uint16).astype(jnp.uint32)
    bu = jax.lax.bitcast_convert_type(
        feat[half:].astype(jnp.bfloat16), jnp.uint16).astype(jnp.uint32)
    return jax.lax.bitcast_convert_type(tu | (bu << 16), jnp.int32)


def _tc_pre_body(xt, z1, wxt, bx1, w0t, al0, ar0, h_o, f_o, el_o, er_o):
    x = xt[:, :]
    h0 = jnp.maximum(_dotT(wxt[:, :], x) + bx1[:, :], 0.0)
    zz = jnp.broadcast_to(z1[:, :], (NH, x.shape[1]))
    hcat = jnp.concatenate([h0, zz], axis=0)
    nrm = jnp.sqrt(jnp.sum(hcat * hcat, axis=0, keepdims=True)) + 1e-6
    h = hcat / nrm
    feat = _dotT(w0t[:, :], h)
    el, er = _elrer(feat, al0[:, :], ar0[:, :])
    h_o[:, :] = h
    f_o[:, :] = _pack_rows(feat)
    el_o[:, :] = el
    er_o[:, :] = er


def _elu(x):
    return jnp.where(x > 0.0, x, jnp.exp(x) - 1.0)


def _tc_mid1_body(o0t, w1t, al1, ar1, h_o, f_o, el_o, er_o):
    h1 = _elu(o0t[:, :])
    feat = _dotT(w1t[:, :], h1)
    el, er = _elrer(feat, al1[:, :], ar1[:, :])
    h_o[:, :] = h1
    f_o[:, :] = _pack_rows(feat)
    el_o[:, :] = el
    er_o[:, :] = er


def _tc_mid2_body(o1t, h1t, w2t, al2, ar2, h_o, f_o, el_o, er_o):
    h2 = _elu(o1t[:, :] + h1t[:, :])
    feat = _dotT(w2t[:, :], h2)
    el, er = _elrer(feat, al2[:, :], ar2[:, :])
    h_o[:, :] = h2
    f_o[:, :] = _pack_rows(feat)
    el_o[:, :] = el
    er_o[:, :] = er


def _tc_fin_body(o2t, h2t, rw2t, lg_o):
    lg_o[:, :] = o2t[:, :] + _dotT(rw2t[:, :], h2t[:, :])


def _col_spec(d):
    return pl.BlockSpec((d, NB), lambda j: (0, j))


def _full_spec(shape):
    return pl.BlockSpec(shape, lambda j: tuple(0 for _ in shape))


@functools.cache
def _tc_pre():
    return pl.pallas_call(
        _tc_pre_body,
        grid=(NPAD // NB,),
        in_specs=[
            _col_spec(IN_DIM),
            _full_spec((NH, 1)), _full_spec((NH, IN_DIM)), _full_spec((NH, 1)),
            _full_spec((D0, D0)), _full_spec((D0, 1)), _full_spec((D0, 1)),
        ],
        out_specs=[_col_spec(D0), _col_spec(D0 // 2), _col_spec(1), _col_spec(1)],
        out_shape=[
            jax.ShapeDtypeStruct((D0, NPAD), jnp.float32),
            jax.ShapeDtypeStruct((D0 // 2, NPAD), jnp.int32),
            jax.ShapeDtypeStruct((1, NPAD), jnp.float32),
            jax.ShapeDtypeStruct((1, NPAD), jnp.float32),
        ],
    )


@functools.cache
def _tc_mid1():
    return pl.pallas_call(
        _tc_mid1_body,
        grid=(NPAD // NB,),
        in_specs=[
            _col_spec(D0),
            _full_spec((D0, D0)), _full_spec((D0, 1)), _full_spec((D0, 1)),
        ],
        out_specs=[_col_spec(D0), _col_spec(D0 // 2), _col_spec(1), _col_spec(1)],
        out_shape=[
            jax.ShapeDtypeStruct((D0, NPAD), jnp.float32),
            jax.ShapeDtypeStruct((D0 // 2, NPAD), jnp.int32),
            jax.ShapeDtypeStruct((1, NPAD), jnp.float32),
            jax.ShapeDtypeStruct((1, NPAD), jnp.float32),
        ],
    )


@functools.cache
def _tc_mid2():
    return pl.pallas_call(
        _tc_mid2_body,
        grid=(NPAD // NB,),
        in_specs=[
            _col_spec(D0), _col_spec(D0),
            _full_spec((NCPAD, D0)), _full_spec((NCPAD, 1)), _full_spec((NCPAD, 1)),
        ],
        out_specs=[_col_spec(D0), _col_spec(NCPAD // 2), _col_spec(1), _col_spec(1)],
        out_shape=[
            jax.ShapeDtypeStruct((D0, NPAD), jnp.float32),
            jax.ShapeDtypeStruct((NCPAD // 2, NPAD), jnp.int32),
            jax.ShapeDtypeStruct((1, NPAD), jnp.float32),
            jax.ShapeDtypeStruct((1, NPAD), jnp.float32),
        ],
    )


@functools.cache
def _tc_fin():
    return pl.pallas_call(
        _tc_fin_body,
        grid=(NPAD // NB,),
        in_specs=[
            _col_spec(NCPAD), _col_spec(D0), _full_spec((NCPAD, D0)),
        ],
        out_specs=_col_spec(NCPAD),
        out_shape=jax.ShapeDtypeStruct((NCPAD, NPAD), jnp.float32),
    )


def kernel(inputs, z, edge_index, Wx, bx, W0, al0, ar0, W1, al1, ar1,
           W2, al2, ar2, resW2):
    src = edge_index[0]
    dst = edge_index[1]

    xt = jnp.pad(inputs, ((0, NPAD - N), (0, 0))).T
    z1 = z[:, None]
    bx1 = bx[:, None]
    wxt = Wx.T
    w0t = W0.T
    w1t = W1.T
    w2t = jnp.pad(W2, ((0, 0), (0, NCPAD - NC))).T
    rw2t = jnp.pad(resW2, ((0, 0), (0, NCPAD - NC))).T
    al0c = al0.reshape(D0, 1)
    ar0c = ar0.reshape(D0, 1)
    al1c = al1.reshape(D0, 1)
    ar1c = ar1.reshape(D0, 1)
    al2c = jnp.pad(al2.reshape(NC, 1), ((0, NCPAD - NC), (0, 0)))
    ar2c = jnp.pad(ar2.reshape(NC, 1), ((0, NCPAD - NC), (0, 0)))

    ht, f0, el0, er0 = _tc_pre()(xt, z1, wxt, bx1, w0t, al0c, ar0c)
    o0, _, _ = _sc_g(D0)(el0.reshape(NPAD), er0.reshape(NPAD), src, dst, f0)

    h1, f1, el1, er1 = _tc_mid1()(o0, w1t, al1c, ar1c)
    o1, _, _ = _sc_g(D0)(el1.reshape(NPAD), er1.reshape(NPAD), src, dst, f1)

    h2, f2, el2, er2 = _tc_mid2()(o1, h1, w2t, al2c, ar2c)
    o2, _, _ = _sc_g(NCPAD)(el2.reshape(NPAD), er2.reshape(NPAD), src, dst, f2)

    lgt = _tc_fin()(o2, h2, rw2t)
    return lgt[:NC, :N].T


# speedup vs baseline: 1.1500x; 1.0148x over previous
import functools

import jax
import jax.numpy as jnp
from jax import lax
from jax.experimental import pallas as pl
from jax.experimental.pallas import tpu as pltpu
from jax.experimental.pallas import tpu_sc as plsc

N = 10000
E = 320000
IN_DIM = 128
NH = 64
D0 = 128
NC = 40
NCPAD = 64

NPAD = 10240
NB = 1024
NCORE = 2
NSUB = 16
NWORK = NCORE * NSUB
LANES = 16
EW = E // NWORK
NSLICE = NPAD // NSUB
EB = 8000
NEB = E // EB


def _mesh():
    return plsc.VectorSubcoreMesh(
        core_axis_name="c", subcore_axis_name="s",
        num_cores=NCORE, num_subcores=NSUB)


_SC_PARAMS = pltpu.CompilerParams(needs_layout_passes=False)


def _zero_f32(ref, n):
    def body(i, _):
        ref[pl.ds(i * LANES, LANES)] = jnp.zeros((LANES,), jnp.float32)
        return None
    lax.fori_loop(0, n // LANES, body, None)


ES = E // NSUB


def _sc_g_body(r_rows, dpad, el_h, er_h, src_h, dst_h, feat_h, out_h, aux_h,
               parts_h, sems, srcb, dstb, pb, elb, erb, denl, redb, acc, *rest):
    frows = rest[:r_rows]
    denf = rest[r_rows]
    c = lax.axis_index("c")
    s = lax.axis_index("s")
    r0 = (c * NSUB + s) * r_rows

    for j in range(r_rows):
        pltpu.async_copy(feat_h.at[r0 + j], frows[j], sems.at[2, 0])

    base = s * ES
    pltpu.async_copy(src_h.at[pl.ds(base, ES)], srcb, sems.at[3, 1])
    pltpu.async_copy(dst_h.at[pl.ds(base, ES)], dstb, sems.at[3, 2])
    pltpu.async_copy(el_h, elb, sems.at[2, 1])
    pltpu.async_copy(er_h, erb, sems.at[2, 2])
    _zero_f32(denl, NPAD)
    pltpu.make_async_copy(src_h.at[pl.ds(base, ES)], srcb, sems.at[3, 1]).wait()
    pltpu.make_async_copy(dst_h.at[pl.ds(base, ES)], dstb, sems.at[3, 2]).wait()
    pltpu.make_async_copy(el_h, elb, sems.at[2, 1]).wait()
    pltpu.make_async_copy(er_h, erb, sems.at[2, 2]).wait()

    @plsc.parallel_loop(0, ES // LANES, unroll=4)
    def edge_chunk(i):
        i16 = pl.ds(i * LANES, LANES)
        s16 = srcb[i16]
        d16 = dstb[i16]
        a = plsc.load_gather(elb, [s16])
        b = plsc.load_gather(erb, [d16])
        e = a + b
        e = jnp.where(e >= 0.0, e, 0.2 * e)
        pv = jnp.exp(e)
        pb[i16] = pv
        plsc.addupdate_scatter(denl, [d16], pv)

    pltpu.sync_copy(denl, parts_h.at[pl.ds((c * NSUB + s) * NPAD, NPAD)])
    plsc.subcore_barrier()

    for t in range(NSUB):
        pltpu.async_copy(
            parts_h.at[pl.ds((c * NSUB + t) * NPAD + s * NSLICE, NSLICE)],
            redb.at[pl.ds(t * NSLICE, NSLICE)], sems.at[3, 0])
    for t in range(NSUB):
        pltpu.make_async_copy(
            parts_h.at[pl.ds((c * NSUB + t) * NPAD + s * NSLICE, NSLICE)],
            redb.at[pl.ds(t * NSLICE, NSLICE)], sems.at[3, 0]).wait()
    _zero_f32(acc, NSLICE)

    def part_add(t, _):
        def vec_add(k, _):
            k16 = pl.ds(k * LANES, LANES)
            acc[k16] = acc[k16] + redb[pl.ds(t * NSLICE + k * LANES, LANES)]
            return None
        lax.fori_loop(0, NSLICE // LANES, vec_add, None)
        return None
    lax.fori_loop(0, NSUB, part_add, None)

    pltpu.sync_copy(acc, denf.at[pl.ds(s * NSLICE, NSLICE)])
    plsc.subcore_barrier()
    pltpu.sync_copy(denf, denl)

    @plsc.parallel_loop(0, ES // LANES, unroll=4)
    def alpha_chunk(i):
        i16 = pl.ds(i * LANES, LANES)
        d16 = dstb[i16]
        dg = plsc.load_gather(denl, [d16])
        pb[i16] = pb[i16] / dg

    pltpu.sync_copy(pb, aux_h.at[pl.ds(c * E + base, ES)])

    for j in range(r_rows):
        pltpu.make_async_copy(feat_h.at[r0 + j], frows[j],
                              sems.at[2, 0]).wait()
    orows = [elb, erb, denl, redb][:2 * r_rows]
    for o in orows:
        _zero_f32(o, NPAD)
    plsc.subcore_barrier()

    def start(b, q):
        eb = b * EB
        qb = q * EB
        pltpu.async_copy(src_h.at[pl.ds(eb, EB)],
                         srcb.at[pl.ds(qb, EB)], sems.at[q, 0])
        pltpu.async_copy(dst_h.at[pl.ds(eb, EB)],
                         dstb.at[pl.ds(qb, EB)], sems.at[q, 1])
        pltpu.async_copy(aux_h.at[pl.ds(c * E + eb, EB)],
                         pb.at[pl.ds(qb, EB)], sems.at[q, 2])

    def wait(b, q):
        eb = b * EB
        qb = q * EB
        pltpu.make_async_copy(src_h.at[pl.ds(eb, EB)],
                              srcb.at[pl.ds(qb, EB)], sems.at[q, 0]).wait()
        pltpu.make_async_copy(dst_h.at[pl.ds(eb, EB)],
                              dstb.at[pl.ds(qb, EB)], sems.at[q, 1]).wait()
        pltpu.make_async_copy(aux_h.at[pl.ds(c * E + eb, EB)],
                              pb.at[pl.ds(qb, EB)], sems.at[q, 2]).wait()

    def compute(q):
        qb = q * EB

        @plsc.parallel_loop(0, EB // LANES, unroll=4)
        def chunk(k):
            i16 = pl.ds(qb + k * LANES, LANES)
            s16 = srcb[i16]
            d16 = dstb[i16]
            al = pb[i16]
            for j in range(r_rows):
                v = plsc.load_gather(frows[j], [s16])
                lo = plsc.bitcast(v << 16, jnp.float32)
                hi = plsc.bitcast(v & jnp.int32(-65536), jnp.float32)
                plsc.addupdate_scatter(orows[2 * j], [d16], lo * al)
                plsc.addupdate_scatter(orows[2 * j + 1], [d16], hi * al)

    start(0, 0)
    start(1, 1)

    def edge_superblock(i, _):
        b0 = i * 2
        wait(b0, 0)
        compute(0)

        @pl.when(b0 + 2 < NEB)
        def _():
            start(b0 + 2, 0)
        wait(b0 + 1, 1)
        compute(1)

        @pl.when(b0 + 3 < NEB)
        def _():
            start(b0 + 3, 1)
        return None
    lax.fori_loop(0, NEB // 2, edge_superblock, None)

    for j in range(r_rows):
        pltpu.sync_copy(orows[2 * j], out_h.at[r0 + j])
        pltpu.sync_copy(orows[2 * j + 1], out_h.at[r0 + j + dpad // 2])


@functools.cache
def _sc_g(dpad):
    r_rows = dpad // 2 // NWORK
    return pl.kernel(
        functools.partial(_sc_g_body, r_rows, dpad),
        out_type=[
            jax.ShapeDtypeStruct((dpad, NPAD), jnp.float32),
            jax.ShapeDtypeStruct((NCORE * E,), jnp.float32),
            jax.ShapeDtypeStruct((NCORE * NSUB * NPAD,), jnp.float32),
        ],
        mesh=_mesh(),
        scratch_types=[
            pltpu.SemaphoreType.DMA((4, 3)),
            pltpu.VMEM((ES,), jnp.int32),
            pltpu.VMEM((ES,), jnp.int32),
            pltpu.VMEM((ES,), jnp.float32),
            pltpu.VMEM((NPAD,), jnp.float32),
            pltpu.VMEM((NPAD,), jnp.float32),
            pltpu.VMEM((NPAD,), jnp.float32),
            pltpu.VMEM((NPAD,), jnp.float32),
            pltpu.VMEM((NSLICE,), jnp.float32),
        ] + [pltpu.VMEM((NPAD,), jnp.int32) for _ in range(r_rows)]
          + [
            pltpu.VMEM_SHARED((NPAD,), jnp.float32),
        ],
        compiler_params=_SC_PARAMS,
    )


_HIGH = jax.lax.Precision.HIGHEST


def _dotT(a, b):
    return jnp.dot(a, b, precision=_HIGH, preferred_element_type=jnp.float32)


def _elrer(feat, alc, arc):
    el = jnp.sum(feat * alc, axis=0, keepdims=True)
    er = jnp.sum(feat * arc, axis=0, keepdims=True)
    return el, er


def _pack_rows(feat):
    half = feat.shape[0] // 2
    tu = jax.lax.bitcast_convert_type(
        feat[:half].astype(jnp.bfloat16), jnp.uint16).astype(jnp.uint32)
    bu = jax.lax.bitcast_convert_type(
        feat[half:].astype(jnp.bfloat16), jnp.uint16).astype(jnp.uint32)
    return jax.lax.bitcast_convert_type(tu | (bu << 16), jnp.int32)


def _tc_pre_body(xt, z1, wxt, bx1, w0t, al0, ar0, h_o, f_o, el_o, er_o):
    x = xt[:, :]
    h0 = jnp.maximum(_dotT(wxt[:, :], x) + bx1[:, :], 0.0)
    zz = jnp.broadcast_to(z1[:, :], (NH, x.shape[1]))
    hcat = jnp.concatenate([h0, zz], axis=0)
    nrm = jnp.sqrt(jnp.sum(hcat * hcat, axis=0, keepdims=True)) + 1e-6
    h = hcat / nrm
    feat = _dotT(w0t[:, :], h)
    el, er = _elrer(feat, al0[:, :], ar0[:, :])
    h_o[:, :] = h
    f_o[:, :] = _pack_rows(feat)
    el_o[:, :] = el
    er_o[:, :] = er


def _elu(x):
    return jnp.where(x > 0.0, x, jnp.exp(x) - 1.0)


def _tc_mid1_body(o0t, w1t, al1, ar1, h_o, f_o, el_o, er_o):
    h1 = _elu(o0t[:, :])
    feat = _dotT(w1t[:, :], h1)
    el, er = _elrer(feat, al1[:, :], ar1[:, :])
    h_o[:, :] = h1
    f_o[:, :] = _pack_rows(feat)
    el_o[:, :] = el
    er_o[:, :] = er


def _tc_mid2_body(o1t, h1t, w2t, al2, ar2, h_o, f_o, el_o, er_o):
    h2 = _elu(o1t[:, :] + h1t[:, :])
    feat = _dotT(w2t[:, :], h2)
    el, er = _elrer(feat, al2[:, :], ar2[:, :])
    h_o[:, :] = h2
    f_o[:, :] = _pack_rows(feat)
    el_o[:, :] = el
    er_o[:, :] = er


def _tc_fin_body(o2t, h2t, rw2t, lg_o):
    lg_o[:, :] = o2t[:, :] + _dotT(rw2t[:, :], h2t[:, :])


def _col_spec(d):
    return pl.BlockSpec((d, NB), lambda j: (0, j))


def _full_spec(shape):
    return pl.BlockSpec(shape, lambda j: tuple(0 for _ in shape))


@functools.cache
def _tc_pre():
    return pl.pallas_call(
        _tc_pre_body,
        grid=(NPAD // NB,),
        in_specs=[
            _col_spec(IN_DIM),
            _full_spec((NH, 1)), _full_spec((NH, IN_DIM)), _full_spec((NH, 1)),
            _full_spec((D0, D0)), _full_spec((D0, 1)), _full_spec((D0, 1)),
        ],
        out_specs=[_col_spec(D0), _col_spec(D0 // 2), _col_spec(1), _col_spec(1)],
        out_shape=[
            jax.ShapeDtypeStruct((D0, NPAD), jnp.float32),
            jax.ShapeDtypeStruct((D0 // 2, NPAD), jnp.int32),
            jax.ShapeDtypeStruct((1, NPAD), jnp.float32),
            jax.ShapeDtypeStruct((1, NPAD), jnp.float32),
        ],
    )


@functools.cache
def _tc_mid1():
    return pl.pallas_call(
        _tc_mid1_body,
        grid=(NPAD // NB,),
        in_specs=[
            _col_spec(D0),
            _full_spec((D0, D0)), _full_spec((D0, 1)), _full_spec((D0, 1)),
        ],
        out_specs=[_col_spec(D0), _col_spec(D0 // 2), _col_spec(1), _col_spec(1)],
        out_shape=[
            jax.ShapeDtypeStruct((D0, NPAD), jnp.float32),
            jax.ShapeDtypeStruct((D0 // 2, NPAD), jnp.int32),
            jax.ShapeDtypeStruct((1, NPAD), jnp.float32),
            jax.ShapeDtypeStruct((1, NPAD), jnp.float32),
        ],
    )


@functools.cache
def _tc_mid2():
    return pl.pallas_call(
        _tc_mid2_body,
        grid=(NPAD // NB,),
        in_specs=[
            _col_spec(D0), _col_spec(D0),
            _full_spec((NCPAD, D0)), _full_spec((NCPAD, 1)), _full_spec((NCPAD, 1)),
        ],
        out_specs=[_col_spec(D0), _col_spec(NCPAD // 2), _col_spec(1), _col_spec(1)],
        out_shape=[
            jax.ShapeDtypeStruct((D0, NPAD), jnp.float32),
            jax.ShapeDtypeStruct((NCPAD // 2, NPAD), jnp.int32),
            jax.ShapeDtypeStruct((1, NPAD), jnp.float32),
            jax.ShapeDtypeStruct((1, NPAD), jnp.float32),
        ],
    )


@functools.cache
def _tc_fin():
    return pl.pallas_call(
        _tc_fin_body,
        grid=(NPAD // NB,),
        in_specs=[
            _col_spec(NCPAD), _col_spec(D0), _full_spec((NCPAD, D0)),
        ],
        out_specs=_col_spec(NCPAD),
        out_shape=jax.ShapeDtypeStruct((NCPAD, NPAD), jnp.float32),
    )


def kernel(inputs, z, edge_index, Wx, bx, W0, al0, ar0, W1, al1, ar1,
           W2, al2, ar2, resW2):
    src = edge_index[0]
    dst = edge_index[1]

    xt = jnp.pad(inputs, ((0, NPAD - N), (0, 0))).T
    z1 = z[:, None]
    bx1 = bx[:, None]
    wxt = Wx.T
    w0t = W0.T
    w1t = W1.T
    w2t = jnp.pad(W2, ((0, 0), (0, NCPAD - NC))).T
    rw2t = jnp.pad(resW2, ((0, 0), (0, NCPAD - NC))).T
    al0c = al0.reshape(D0, 1)
    ar0c = ar0.reshape(D0, 1)
    al1c = al1.reshape(D0, 1)
    ar1c = ar1.reshape(D0, 1)
    al2c = jnp.pad(al2.reshape(NC, 1), ((0, NCPAD - NC), (0, 0)))
    ar2c = jnp.pad(ar2.reshape(NC, 1), ((0, NCPAD - NC), (0, 0)))

    ht, f0, el0, er0 = _tc_pre()(xt, z1, wxt, bx1, w0t, al0c, ar0c)
    o0, _, _ = _sc_g(D0)(el0.reshape(NPAD), er0.reshape(NPAD), src, dst, f0)

    h1, f1, el1, er1 = _tc_mid1()(o0, w1t, al1c, ar1c)
    o1, _, _ = _sc_g(D0)(el1.reshape(NPAD), er1.reshape(NPAD), src, dst, f1)

    h2, f2, el2, er2 = _tc_mid2()(o1, h1, w2t, al2c, ar2c)
    o2, _, _ = _sc_g(NCPAD)(el2.reshape(NPAD), er2.reshape(NPAD), src, dst, f2)

    lgt = _tc_fin()(o2, h2, rw2t)
    return lgt[:NC, :N].T
